# baseline (speedup 1.0000x reference)
import functools

import jax
import jax.numpy as jnp
from jax import lax
from jax.experimental import pallas as pl
from jax.experimental.pallas import tpu as pltpu
from jax.experimental.pallas import tpu_sc as plsc

_EPS = 1e-12

_NC = 2
_NS = 16
_NW = _NC * _NS


def _sc_gather(table, idx):
    BS = idx.shape[0]
    H = table.shape[1]
    b_per_w = BS // _NW
    CH = 64
    n_ch = b_per_w // CH

    mesh = plsc.VectorSubcoreMesh(core_axis_name="c", subcore_axis_name="s")

    @functools.partial(
        pl.kernel,
        mesh=mesh,
        out_type=jax.ShapeDtypeStruct((BS, H), jnp.float32),
        scratch_types=[
            pltpu.VMEM((CH,), jnp.int32),
            pltpu.VMEM((CH, H), jnp.float32),
            pltpu.SemaphoreType.DMA,
        ],
    )
    def k(idx_hbm, table_hbm, out_hbm, idx_v, rows_v, sem):
        wid = lax.axis_index("s") * _NC + lax.axis_index("c")
        base = wid * b_per_w

        def body(i, carry):
            off = base + i * CH
            pltpu.sync_copy(idx_hbm.at[pl.ds(off, CH)], idx_v)
            pltpu.async_copy(table_hbm.at[idx_v], rows_v, sem).wait()
            pltpu.sync_copy(rows_v, out_hbm.at[pl.ds(off, CH)])
            return carry

        lax.fori_loop(0, n_ch, body, 0)

    return k(idx, table)


def _tc_fused(gathered, tt_ids, ve, vtt_ids, pos_s, tte, vtte, vpos0,
              w, b2, gam, bet, BB):
    B, S, H = gathered.shape
    V, VD = ve.shape[1], ve.shape[2]
    grid = (B // BB,)
    ttf = tt_ids.astype(jnp.float32).reshape(B, S, 1)
    vttf = vtt_ids.astype(jnp.float32).reshape(B, V, 1)

    def body(g_ref, tt_ref, ve_ref, vtt_ref, pos_ref, tte_ref, vtte_ref,
             vpos_ref, w_ref, b_ref, gam_ref, bet_ref, o_ref):
        g = gam_ref[0, :]
        be = bet_ref[0, :]

        def ln(x):
            mu = jnp.mean(x, axis=-1, keepdims=True)
            xc = x - mu
            var = jnp.mean(xc * xc, axis=-1, keepdims=True)
            return xc * lax.rsqrt(var + _EPS) * g[None, :] + be[None, :]

        t0 = tte_ref[0, :]
        t1 = tte_ref[1, :]
        v0 = vtte_ref[0, :] + vpos_ref[0, :] + b_ref[0, :]
        v1 = vtte_ref[1, :] + vpos_ref[0, :] + b_ref[0, :]
        pbase = pos_ref[...] + t0[None, :]
        tdiff = (t1 - t0)[None, None, :]
        text = g_ref[...] + pbase[None, :, :] + tt_ref[...] * tdiff
        o_ref[:, :S, :] = ln(text)
        vdiff = (v1 - v0)[None, :]
        for bb in range(BB):
            vm = lax.dot_general(ve_ref[bb], w_ref[...],
                                 (((1,), (1,)), ((), ())),
                                 preferred_element_type=jnp.float32)
            vis = vm + v0[None, :] + vtt_ref[bb] * vdiff
            o_ref[bb, S:, :] = ln(vis)

    return pl.pallas_call(
        body,
        grid=grid,
        in_specs=[
            pl.BlockSpec((BB, S, H), lambda i: (i, 0, 0)),
            pl.BlockSpec((BB, S, 1), lambda i: (i, 0, 0)),
            pl.BlockSpec((BB, V, VD), lambda i: (i, 0, 0)),
            pl.BlockSpec((BB, V, 1), lambda i: (i, 0, 0)),
            pl.BlockSpec((S, H), lambda i: (0, 0)),
            pl.BlockSpec((2, H), lambda i: (0, 0)),
            pl.BlockSpec((2, H), lambda i: (0, 0)),
            pl.BlockSpec((1, H), lambda i: (0, 0)),
            pl.BlockSpec((H, VD), lambda i: (0, 0)),
            pl.BlockSpec((1, H), lambda i: (0, 0)),
            pl.BlockSpec((1, H), lambda i: (0, 0)),
            pl.BlockSpec((1, H), lambda i: (0, 0)),
        ],
        out_specs=pl.BlockSpec((BB, S + V, H), lambda i: (i, 0, 0)),
        out_shape=jax.ShapeDtypeStruct((B, S + V, H), jnp.float32),
    )(gathered, ttf, ve, vttf, pos_s, tte, vtte, vpos0, w, b2, gam, bet)


def kernel(input_ids, token_type_ids, visual_embeds, visual_token_type_ids,
           word_emb, pos_emb, tok_type_emb, vis_tok_type_emb, vis_pos_emb,
           vproj_w, vproj_b, ln_gamma, ln_beta):
    B, S = input_ids.shape
    H = word_emb.shape[1]
    gathered = _sc_gather(word_emb, input_ids.reshape(-1)).reshape(B, S, H)
    return _tc_fused(
        gathered, token_type_ids, visual_embeds, visual_token_type_ids,
        pos_emb[:S], tok_type_emb, vis_tok_type_emb, vis_pos_emb[0:1],
        vproj_w, vproj_b.reshape(1, H), ln_gamma.reshape(1, H),
        ln_beta.reshape(1, H), BB=8)


# speedup vs baseline: 1.2125x; 1.2125x over previous
import functools

import jax
import jax.numpy as jnp
from jax import lax
from jax.experimental import pallas as pl
from jax.experimental.pallas import tpu as pltpu
from jax.experimental.pallas import tpu_sc as plsc

_EPS = 1e-12

_NC = 2
_NS = 16
_NW = _NC * _NS


def _sc_gather(table, idx):
    BS = idx.shape[0]
    H = table.shape[1]
    b_per_w = BS // _NW
    CH = 64
    n_ch = b_per_w // CH

    mesh = plsc.VectorSubcoreMesh(core_axis_name="c", subcore_axis_name="s")

    @functools.partial(
        pl.kernel,
        mesh=mesh,
        out_type=jax.ShapeDtypeStruct((BS, H), jnp.float32),
        scratch_types=[
            pltpu.VMEM((CH,), jnp.int32),
            pltpu.VMEM((CH, H), jnp.float32),
            pltpu.SemaphoreType.DMA,
        ],
    )
    def k(idx_hbm, table_hbm, out_hbm, idx_v, rows_v, sem):
        wid = lax.axis_index("s") * _NC + lax.axis_index("c")
        base = wid * b_per_w

        def body(i, carry):
            off = base + i * CH
            pltpu.sync_copy(idx_hbm.at[pl.ds(off, CH)], idx_v)
            pltpu.async_copy(table_hbm.at[idx_v], rows_v, sem).wait()
            pltpu.sync_copy(rows_v, out_hbm.at[pl.ds(off, CH)])
            return carry

        lax.fori_loop(0, n_ch, body, 0)

    return k(idx, table)


def _tc_fused(gathered, tt_ids, ve, vtt_ids, pos_s, tte, vtte, vpos0,
              w, b2, gam, bet, BB):
    B, S, H = gathered.shape
    V, VD = ve.shape[1], ve.shape[2]
    grid = (B // BB,)
    ttf = tt_ids.astype(jnp.float32).reshape(B, S, 1)
    vttf = vtt_ids.astype(jnp.float32).reshape(B * V, 1)
    ve2 = ve.reshape(B * V, VD)
    wt = w.T.astype(jnp.bfloat16)

    def body(g_ref, tt_ref, ve_ref, vtt_ref, pos_ref, tte_ref, vtte_ref,
             vpos_ref, w_ref, b_ref, gam_ref, bet_ref, o_ref):
        g = gam_ref[0, :]
        be = bet_ref[0, :]

        def ln(x):
            mu = jnp.mean(x, axis=-1, keepdims=True)
            xc = x - mu
            var = jnp.mean(xc * xc, axis=-1, keepdims=True)
            return xc * lax.rsqrt(var + _EPS) * g[None, :] + be[None, :]

        t0 = tte_ref[0, :]
        t1 = tte_ref[1, :]
        v0 = vtte_ref[0, :] + vpos_ref[0, :] + b_ref[0, :]
        v1 = vtte_ref[1, :] + vpos_ref[0, :] + b_ref[0, :]
        pbase = pos_ref[...] + t0[None, :]
        tdiff = (t1 - t0)[None, None, :]
        text = g_ref[...] + pbase[None, :, :] + tt_ref[...] * tdiff
        o_ref[:, :S, :] = ln(text)
        veb = ve_ref[...].astype(jnp.bfloat16)
        vm = lax.dot_general(veb, w_ref[...], (((1,), (0,)), ((), ())),
                             preferred_element_type=jnp.float32)
        vis = ln(vm + v0[None, :] + vtt_ref[...] * (v1 - v0)[None, :])
        for bb in range(BB):
            o_ref[bb, S:, :] = vis[bb * V:(bb + 1) * V, :]

    return pl.pallas_call(
        body,
        grid=grid,
        in_specs=[
            pl.BlockSpec((BB, S, H), lambda i: (i, 0, 0)),
            pl.BlockSpec((BB, S, 1), lambda i: (i, 0, 0)),
            pl.BlockSpec((BB * V, VD), lambda i: (i, 0)),
            pl.BlockSpec((BB * V, 1), lambda i: (i, 0)),
            pl.BlockSpec((S, H), lambda i: (0, 0)),
            pl.BlockSpec((2, H), lambda i: (0, 0)),
            pl.BlockSpec((2, H), lambda i: (0, 0)),
            pl.BlockSpec((1, H), lambda i: (0, 0)),
            pl.BlockSpec((VD, H), lambda i: (0, 0)),
            pl.BlockSpec((1, H), lambda i: (0, 0)),
            pl.BlockSpec((1, H), lambda i: (0, 0)),
            pl.BlockSpec((1, H), lambda i: (0, 0)),
        ],
        out_specs=pl.BlockSpec((BB, S + V, H), lambda i: (i, 0, 0)),
        out_shape=jax.ShapeDtypeStruct((B, S + V, H), jnp.float32),
    )(gathered, ttf, ve2, vttf, pos_s, tte, vtte, vpos0, wt, b2, gam, bet)


def kernel(input_ids, token_type_ids, visual_embeds, visual_token_type_ids,
           word_emb, pos_emb, tok_type_emb, vis_tok_type_emb, vis_pos_emb,
           vproj_w, vproj_b, ln_gamma, ln_beta):
    B, S = input_ids.shape
    H = word_emb.shape[1]
    gathered = _sc_gather(word_emb, input_ids.reshape(-1)).reshape(B, S, H)
    return _tc_fused(
        gathered, token_type_ids, visual_embeds, visual_token_type_ids,
        pos_emb[:S], tok_type_emb, vis_tok_type_emb, vis_pos_emb[0:1],
        vproj_w, vproj_b.reshape(1, H), ln_gamma.reshape(1, H),
        ln_beta.reshape(1, H), BB=8)


# speedup vs baseline: 1.2545x; 1.0346x over previous
import functools

import jax
import jax.numpy as jnp
from jax import lax
from jax.experimental import pallas as pl
from jax.experimental.pallas import tpu as pltpu
from jax.experimental.pallas import tpu_sc as plsc

_EPS = 1e-12

_NC = 2
_NS = 16
_NW = _NC * _NS


def _sc_gather(table, idx):
    BS = idx.shape[0]
    H = table.shape[1]
    b_per_w = BS // _NW
    CH = 80
    n_ch = b_per_w // CH
    n2 = n_ch // 2

    mesh = plsc.VectorSubcoreMesh(core_axis_name="c", subcore_axis_name="s")

    @functools.partial(
        pl.kernel,
        mesh=mesh,
        out_type=jax.ShapeDtypeStruct((BS, H), jnp.float32),
        scratch_types=[
            pltpu.VMEM((b_per_w,), jnp.int32),
            pltpu.VMEM((CH, H), jnp.float32),
            pltpu.VMEM((CH, H), jnp.float32),
            pltpu.SemaphoreType.DMA,
            pltpu.SemaphoreType.DMA,
            pltpu.SemaphoreType.DMA,
            pltpu.SemaphoreType.DMA,
        ],
    )
    def k(idx_hbm, table_hbm, out_hbm, idx_v, buf0, buf1, gs0, gs1, ws0, ws1):
        wid = lax.axis_index("s") * _NC + lax.axis_index("c")
        base = wid * b_per_w
        pltpu.sync_copy(idx_hbm.at[pl.ds(base, b_per_w)], idx_v)

        def g_start(c, buf, sem):
            pltpu.async_copy(table_hbm.at[idx_v.at[pl.ds(c * CH, CH)]],
                             buf, sem)

        def g_wait(buf, sem):
            pltpu.make_async_copy(table_hbm.at[idx_v.at[pl.ds(0, CH)]],
                                  buf, sem).wait()

        def w_start(c, buf, sem):
            pltpu.async_copy(buf, out_hbm.at[pl.ds(base + c * CH, CH)], sem)

        def w_wait(buf, sem):
            pltpu.make_async_copy(buf, out_hbm.at[pl.ds(base, CH)],
                                  sem).wait()

        g_start(0, buf0, gs0)

        def body(jj, carry):
            c0 = 2 * jj
            g_wait(buf0, gs0)
            g_start(c0 + 1, buf1, gs1)
            w_start(c0, buf0, ws0)
            g_wait(buf1, gs1)
            w_wait(buf0, ws0)
            g_start(c0 + 2, buf0, gs0)
            w_start(c0 + 1, buf1, ws1)
            w_wait(buf1, ws1)
            return carry

        lax.fori_loop(0, n2 - 1, body, 0)

        c0 = n_ch - 2
        g_wait(buf0, gs0)
        g_start(c0 + 1, buf1, gs1)
        w_start(c0, buf0, ws0)
        g_wait(buf1, gs1)
        w_wait(buf0, ws0)
        w_start(c0 + 1, buf1, ws1)
        w_wait(buf1, ws1)

    return k(idx, table)


def _tc_fused(gathered, tt_ids, ve, vtt_ids, pos_s, tte, vtte, vpos0,
              w, b2, gam, bet, BB):
    B, S, H = gathered.shape
    V, VD = ve.shape[1], ve.shape[2]
    grid = (B // BB,)
    ttf = tt_ids.astype(jnp.float32).reshape(B, S, 1)
    vttf = vtt_ids.astype(jnp.float32).reshape(B * V, 1)
    ve2 = ve.reshape(B * V, VD)
    wt = w.T.astype(jnp.bfloat16)

    def body(g_ref, tt_ref, ve_ref, vtt_ref, pos_ref, tte_ref, vtte_ref,
             vpos_ref, w_ref, b_ref, gam_ref, bet_ref, o_ref):
        g = gam_ref[0, :]
        be = bet_ref[0, :]

        def ln(x):
            mu = jnp.mean(x, axis=-1, keepdims=True)
            xc = x - mu
            var = jnp.mean(xc * xc, axis=-1, keepdims=True)
            return xc * lax.rsqrt(var + _EPS) * g[None, :] + be[None, :]

        t0 = tte_ref[0, :]
        t1 = tte_ref[1, :]
        v0 = vtte_ref[0, :] + vpos_ref[0, :] + b_ref[0, :]
        v1 = vtte_ref[1, :] + vpos_ref[0, :] + b_ref[0, :]
        pbase = pos_ref[...] + t0[None, :]
        tdiff = (t1 - t0)[None, None, :]
        text = g_ref[...] + pbase[None, :, :] + tt_ref[...] * tdiff
        o_ref[:, :S, :] = ln(text)
        veb = ve_ref[...].astype(jnp.bfloat16)
        vm = lax.dot_general(veb, w_ref[...], (((1,), (0,)), ((), ())),
                             preferred_element_type=jnp.float32)
        vis = ln(vm + v0[None, :] + vtt_ref[...] * (v1 - v0)[None, :])
        for bb in range(BB):
            o_ref[bb, S:, :] = vis[bb * V:(bb + 1) * V, :]

    return pl.pallas_call(
        body,
        grid=grid,
        in_specs=[
            pl.BlockSpec((BB, S, H), lambda i: (i, 0, 0)),
            pl.BlockSpec((BB, S, 1), lambda i: (i, 0, 0)),
            pl.BlockSpec((BB * V, VD), lambda i: (i, 0)),
            pl.BlockSpec((BB * V, 1), lambda i: (i, 0)),
            pl.BlockSpec((S, H), lambda i: (0, 0)),
            pl.BlockSpec((2, H), lambda i: (0, 0)),
            pl.BlockSpec((2, H), lambda i: (0, 0)),
            pl.BlockSpec((1, H), lambda i: (0, 0)),
            pl.BlockSpec((VD, H), lambda i: (0, 0)),
            pl.BlockSpec((1, H), lambda i: (0, 0)),
            pl.BlockSpec((1, H), lambda i: (0, 0)),
            pl.BlockSpec((1, H), lambda i: (0, 0)),
        ],
        out_specs=pl.BlockSpec((BB, S + V, H), lambda i: (i, 0, 0)),
        out_shape=jax.ShapeDtypeStruct((B, S + V, H), jnp.float32),
    )(gathered, ttf, ve2, vttf, pos_s, tte, vtte, vpos0, wt, b2, gam, bet)


def kernel(input_ids, token_type_ids, visual_embeds, visual_token_type_ids,
           word_emb, pos_emb, tok_type_emb, vis_tok_type_emb, vis_pos_emb,
           vproj_w, vproj_b, ln_gamma, ln_beta):
    B, S = input_ids.shape
    H = word_emb.shape[1]
    gathered = _sc_gather(word_emb, input_ids.reshape(-1)).reshape(B, S, H)
    return _tc_fused(
        gathered, token_type_ids, visual_embeds, visual_token_type_ids,
        pos_emb[:S], tok_type_emb, vis_tok_type_emb, vis_pos_emb[0:1],
        vproj_w, vproj_b.reshape(1, H), ln_gamma.reshape(1, H),
        ln_beta.reshape(1, H), BB=8)
